# baseline (speedup 1.0000x reference)
import jax
import jax.numpy as jnp
from jax import lax
from jax.experimental import pallas as pl
from jax.experimental.pallas import tpu as pltpu
from jax.experimental.pallas import tpu_sc as plsc

BATCH = 16384
HIST = 50
EMBED_DIM = 64
NUM_CORES = 2
NUM_SUBCORES = 16
NW = NUM_CORES * NUM_SUBCORES
TILE_B = 128
JTILES = BATCH // TILE_B
NTILE = HIST * JTILES
PER_W = NTILE // NW
LANES = 16


def _make_kernel():
    mesh = plsc.VectorSubcoreMesh(
        core_axis_name="c", subcore_axis_name="s",
        num_cores=NUM_CORES, num_subcores=NUM_SUBCORES)

    @pl.kernel(
        out_type=jax.ShapeDtypeStruct(
            (HIST, EMBED_DIM // 8, JTILES, 8, TILE_B), jnp.float32),
        mesh=mesh,
        scratch_types=[
            pltpu.VMEM((PER_W, TILE_B), jnp.int32),
            pltpu.VMEM((TILE_B, EMBED_DIM), jnp.float32),
            pltpu.VMEM((TILE_B, EMBED_DIM), jnp.float32),
            pltpu.VMEM((8, 8, TILE_B), jnp.float32),
            pltpu.VMEM((8, 8, TILE_B), jnp.float32),
            pltpu.SemaphoreType.DMA,
            pltpu.SemaphoreType.DMA,
            pltpu.SemaphoreType.DMA,
            pltpu.SemaphoreType.DMA,
        ],
        compiler_params=pltpu.CompilerParams(
            use_tc_tiling_on_sc=False, needs_layout_passes=False),
    )
    def gather_kernel(ids_hbm, table_hbm, out_hbm, idx_v,
                      rows0, rows1, t30, t31, gs0, gs1, ws0, ws1):
        rows = [rows0, rows1]
        t3 = [t30, t31]
        gs = [gs0, gs1]
        ws = [ws0, ws1]
        wid = lax.axis_index("s") * NUM_CORES + lax.axis_index("c")
        kbase = wid * PER_W
        pltpu.sync_copy(ids_hbm.at[wid], idx_v)

        iota = lax.iota(jnp.int32, LANES)
        bb_idx = [iota + m * LANES for m in range(TILE_B // LANES)]

        def fire(kk, b):
            pltpu.async_copy(table_hbm.at[idx_v.at[kk]], rows[b], gs[b])

        def transpose(b):
            for g in range(8):
                for cc in range(8):
                    c_idx = jnp.full((LANES,), g * 8 + cc, jnp.int32)
                    for m in range(TILE_B // LANES):
                        v = plsc.load_gather(rows[b], [bb_idx[m], c_idx])
                        t3[b][g, cc, pl.ds(m * LANES, LANES)] = v

        for b in range(2):
            fire(b, b)

        def outer(o, _):
            for b in range(2):
                kk = o * 2 + b
                k = kbase + kk
                t = k >> 7
                j = k & (JTILES - 1)
                pltpu.make_async_copy(
                    table_hbm.at[pl.ds(0, TILE_B)], rows[b], gs[b]).wait()

                @pl.when(o > 0)
                def _():
                    pltpu.make_async_copy(
                        t3[b], out_hbm.at[0, :, 0], ws[b]).wait()

                transpose(b)
                pltpu.async_copy(t3[b], out_hbm.at[t, :, j], ws[b])

                @pl.when(o < PER_W // 2 - 1)
                def _():
                    fire(kk + 2, b)
            return ()

        lax.fori_loop(0, PER_W // 2, outer, (), unroll=False)
        for b in range(2):
            pltpu.make_async_copy(t3[b], out_hbm.at[0, :, 0], ws[b]).wait()

    return gather_kernel


_gather = _make_kernel()


def kernel(input_ids, embedding_table):
    ids_t = jnp.transpose(input_ids.astype(jnp.int32)).reshape(NW, PER_W, TILE_B)
    buf = _gather(ids_t, embedding_table)
    return jnp.transpose(buf, (2, 4, 0, 1, 3)).reshape(BATCH, HIST, EMBED_DIM)


# speedup vs baseline: 1.5506x; 1.5506x over previous
import jax
import jax.numpy as jnp
from jax import lax
from jax.experimental import pallas as pl
from jax.experimental.pallas import tpu as pltpu
from jax.experimental.pallas import tpu_sc as plsc

BATCH = 16384
HIST = 50
EMBED_DIM = 64
NUM_CORES = 2
NUM_SUBCORES = 16
NW = NUM_CORES * NUM_SUBCORES
TILE_B = 128
JTILES = BATCH // TILE_B
NTILE = HIST * JTILES
PER_W = NTILE // NW
LANES = 16


def _make_kernel():
    mesh = plsc.VectorSubcoreMesh(
        core_axis_name="c", subcore_axis_name="s",
        num_cores=NUM_CORES, num_subcores=NUM_SUBCORES)

    @pl.kernel(
        out_type=jax.ShapeDtypeStruct(
            (HIST, EMBED_DIM // 8, JTILES, 8, TILE_B), jnp.float32),
        mesh=mesh,
        scratch_types=[
            pltpu.VMEM((PER_W, TILE_B), jnp.int32),
            pltpu.VMEM((TILE_B, EMBED_DIM), jnp.float32),
            pltpu.VMEM((TILE_B, EMBED_DIM), jnp.float32),
            pltpu.VMEM((8, 8, TILE_B), jnp.float32),
            pltpu.VMEM((8, 8, TILE_B), jnp.float32),
            pltpu.SemaphoreType.DMA,
            pltpu.SemaphoreType.DMA,
            pltpu.SemaphoreType.DMA,
            pltpu.SemaphoreType.DMA,
        ],
        compiler_params=pltpu.CompilerParams(
            use_tc_tiling_on_sc=False, needs_layout_passes=False),
    )
    def gather_kernel(ids_hbm, table_hbm, out_hbm, idx_v,
                      rows0, rows1, t30, t31, gs0, gs1, ws0, ws1):
        rows = [rows0, rows1]
        t3 = [t30, t31]
        gs = [gs0, gs1]
        ws = [ws0, ws1]
        wid = lax.axis_index("s") * NUM_CORES + lax.axis_index("c")
        kbase = wid * PER_W
        pltpu.sync_copy(ids_hbm.at[wid], idx_v)

        iota = lax.iota(jnp.int32, LANES)
        g_idx = [(iota + cb * LANES) >> 3 for cb in range(EMBED_DIM // LANES)]
        cc_idx = [(iota + cb * LANES) & 7 for cb in range(EMBED_DIM // LANES)]

        def fire(kk, b):
            pltpu.async_copy(table_hbm.at[idx_v.at[kk]], rows[b], gs[b])

        def transpose(b):
            @plsc.parallel_loop(0, TILE_B, step=1, unroll=8)
            def _(bb):
                bb_vec = jnp.full((LANES,), bb, jnp.int32)
                for cb in range(EMBED_DIM // LANES):
                    v = rows[b][bb, pl.ds(cb * LANES, LANES)]
                    plsc.store_scatter(t3[b], [g_idx[cb], cc_idx[cb], bb_vec], v)

        for b in range(2):
            fire(b, b)

        def outer(o, _):
            for b in range(2):
                kk = o * 2 + b
                k = kbase + kk
                t = k >> 7
                j = k & (JTILES - 1)
                pltpu.make_async_copy(
                    table_hbm.at[pl.ds(0, TILE_B)], rows[b], gs[b]).wait()

                @pl.when(o > 0)
                def _():
                    pltpu.make_async_copy(
                        t3[b], out_hbm.at[0, :, 0], ws[b]).wait()

                transpose(b)
                pltpu.async_copy(t3[b], out_hbm.at[t, :, j], ws[b])

                @pl.when(o < PER_W // 2 - 1)
                def _():
                    fire(kk + 2, b)
            return ()

        lax.fori_loop(0, PER_W // 2, outer, (), unroll=False)
        for b in range(2):
            pltpu.make_async_copy(t3[b], out_hbm.at[0, :, 0], ws[b]).wait()

    return gather_kernel


_gather = _make_kernel()


def kernel(input_ids, embedding_table):
    ids_t = jnp.transpose(input_ids.astype(jnp.int32)).reshape(NW, PER_W, TILE_B)
    buf = _gather(ids_t, embedding_table)
    return jnp.transpose(buf, (2, 4, 0, 1, 3)).reshape(BATCH, HIST, EMBED_DIM)
